# baseline (speedup 1.0000x reference)
import functools
import jax
import jax.numpy as jnp
from jax.experimental import pallas as pl
from jax.experimental.pallas import tpu as pltpu

_E = 8
_TOPK = 2

_TM = 1024
_TO = 512


def _dense_moe_body(x_ref, xb16_ref, wg_ref, we_ref, be_ref, out_ref, wd_ref):
    ot = pl.program_id(1)
    e = pl.program_id(2)

    @pl.when(jnp.logical_and(ot == 0, e == 0))
    def _gate():
        xb = x_ref[...]
        logits = jax.lax.dot_general(
            xb, wg_ref[...], (((1,), (1,)), ((), ())),
            preferred_element_type=jnp.float32)
        m = jnp.max(logits, axis=1, keepdims=True)
        ex = jnp.exp(logits - m)
        p = ex / jnp.sum(ex, axis=1, keepdims=True)
        lanes = jax.lax.broadcasted_iota(jnp.int32, p.shape, 1)
        m1 = jnp.max(p, axis=1, keepdims=True)
        i1 = jnp.min(jnp.where(p == m1, lanes, _E), axis=1, keepdims=True)
        p2 = jnp.where(lanes == i1, -jnp.inf, p)
        m2 = jnp.max(p2, axis=1, keepdims=True)
        i2 = jnp.min(jnp.where(p2 == m2, lanes, _E), axis=1, keepdims=True)
        wd = jnp.where(lanes == i1, m1, 0.0) + jnp.where(lanes == i2, m2, 0.0)
        wd_ref[...] = wd

    @pl.when(e == 0)
    def _zero():
        out_ref[...] = jnp.zeros_like(out_ref)

    lanes = jax.lax.broadcasted_iota(jnp.int32, wd_ref.shape, 1)
    w_e = jnp.sum(jnp.where(lanes == e, wd_ref[...], 0.0), axis=1,
                  keepdims=True)
    acc = jax.lax.dot_general(
        xb16_ref[...], we_ref[0], (((1,), (1,)), ((), ())),
        preferred_element_type=jnp.float32)
    acc = acc + be_ref[0]
    out_ref[...] += w_e * acc


def _dense_moe(xf, xf16, Wg, We16, be):
    n_tok, in_dim = xf.shape
    out_dim = We16.shape[1]
    grid = (n_tok // _TM, out_dim // _TO, _E)
    return pl.pallas_call(
        _dense_moe_body,
        grid=grid,
        in_specs=[
            pl.BlockSpec((_TM, in_dim), lambda mt, ot, e: (mt, 0)),
            pl.BlockSpec((_TM, in_dim), lambda mt, ot, e: (mt, 0)),
            pl.BlockSpec((_E, in_dim), lambda mt, ot, e: (0, 0)),
            pl.BlockSpec((1, _TO, in_dim), lambda mt, ot, e: (e, ot, 0)),
            pl.BlockSpec((1, 1, _TO), lambda mt, ot, e: (e, 0, ot)),
        ],
        out_specs=pl.BlockSpec((_TM, _TO), lambda mt, ot, e: (mt, ot)),
        out_shape=jax.ShapeDtypeStruct((n_tok, out_dim), jnp.float32),
        scratch_shapes=[pltpu.VMEM((_TM, _E), jnp.float32)],
    )(xf, xf16, Wg, We16, be)


def kernel(x, Wg, We, be):
    B, in_len, n_vars = x.shape
    xf = jnp.transpose(x, (0, 2, 1)).reshape(B * n_vars, in_len)
    xf16 = xf.astype(jnp.bfloat16)
    We16 = We.astype(jnp.bfloat16)
    be3 = be.reshape(_E, 1, be.shape[-1])
    out = _dense_moe(xf, xf16, Wg, We16, be3)
    out_dim = We.shape[1]
    return jnp.transpose(out.reshape(B, n_vars, out_dim), (0, 2, 1))


# speedup vs baseline: 1.4137x; 1.4137x over previous
import functools
import jax
import jax.numpy as jnp
from jax import lax
from jax.experimental import pallas as pl
from jax.experimental.pallas import tpu as pltpu
from jax.experimental.pallas import tpu_sc as plsc

_E = 8
_T = 256
_GTM = 512
_NW = 32
_SUB = 16


def _gating_body(x_ref, wg_ref, w1_ref, w2_ref, i1_ref, i2_ref,
                 r1_ref, r2_ref, cnt_ref, cacc_ref):
    pid = pl.program_id(0)

    @pl.when(pid == 0)
    def _init():
        cacc_ref[...] = jnp.zeros_like(cacc_ref)

    xb = x_ref[...]
    logits = lax.dot_general(xb, wg_ref[...], (((1,), (1,)), ((), ())),
                             preferred_element_type=jnp.float32)
    m = jnp.max(logits, axis=1, keepdims=True)
    ex = jnp.exp(logits - m)
    p = ex / jnp.sum(ex, axis=1, keepdims=True)
    lanes = lax.broadcasted_iota(jnp.int32, p.shape, 1)
    m1 = jnp.max(p, axis=1, keepdims=True)
    i1 = jnp.min(jnp.where(p == m1, lanes, _E), axis=1, keepdims=True)
    p2 = jnp.where(lanes == i1, -jnp.inf, p)
    m2 = jnp.max(p2, axis=1, keepdims=True)
    i2 = jnp.min(jnp.where(p2 == m2, lanes, _E), axis=1, keepdims=True)

    oh1 = (lanes == i1).astype(jnp.float32)
    oh2 = (lanes == i2).astype(jnp.float32)
    ohc = oh1 + oh2
    rows = lax.broadcasted_iota(jnp.int32, (_GTM, _GTM), 0)
    cols = lax.broadcasted_iota(jnp.int32, (_GTM, _GTM), 1)
    tril = (cols < rows).astype(jnp.float32)
    ranks = lax.dot_general(tril, ohc, (((1,), (0,)), ((), ())),
                            preferred_element_type=jnp.float32)
    ranks = ranks + cacc_ref[...]
    r1 = jnp.sum(ranks * oh1, axis=1, keepdims=True)
    r2 = jnp.sum(ranks * oh2, axis=1, keepdims=True)

    w1_ref[...] = m1
    w2_ref[...] = m2
    i1_ref[...] = i1
    i2_ref[...] = i2
    r1_ref[...] = r1.astype(jnp.int32)
    r2_ref[...] = r2.astype(jnp.int32)
    cacc_ref[...] += jnp.sum(ohc, axis=0, keepdims=True)
    cnt_ref[...] = cacc_ref[...]


def _gating(xf, Wg):
    n_tok, in_dim = xf.shape
    grid = (n_tok // _GTM,)
    f32 = jnp.float32
    i32 = jnp.int32
    outs = [jax.ShapeDtypeStruct((n_tok, 1), f32),
            jax.ShapeDtypeStruct((n_tok, 1), f32),
            jax.ShapeDtypeStruct((n_tok, 1), i32),
            jax.ShapeDtypeStruct((n_tok, 1), i32),
            jax.ShapeDtypeStruct((n_tok, 1), i32),
            jax.ShapeDtypeStruct((n_tok, 1), i32),
            jax.ShapeDtypeStruct((1, _E), f32)]
    tok_spec = pl.BlockSpec((_GTM, 1), lambda i: (i, 0))
    return pl.pallas_call(
        _gating_body,
        grid=grid,
        in_specs=[pl.BlockSpec((_GTM, in_dim), lambda i: (i, 0)),
                  pl.BlockSpec((_E, in_dim), lambda i: (0, 0))],
        out_specs=[tok_spec, tok_spec, tok_spec, tok_spec, tok_spec, tok_spec,
                   pl.BlockSpec((1, _E), lambda i: (0, 0))],
        out_shape=outs,
        scratch_shapes=[pltpu.VMEM((1, _E), f32)],
    )(xf, Wg)


def _finalize_body(cnt_ref, i1_ref, i2_ref, r1_ref, r2_ref,
                   s1_ref, s2_ref, gid_ref):
    c = cnt_ref[...].astype(jnp.int32)
    aligned = ((c + (_T - 1)) // _T) * _T
    rows = lax.broadcasted_iota(jnp.int32, (_E, _E), 0)
    cols = lax.broadcasted_iota(jnp.int32, (_E, _E), 1)
    tril = (cols < rows).astype(jnp.float32)
    off = lax.dot_general(tril, aligned.astype(jnp.float32),
                          (((1,), (0,)), ((), ())),
                          preferred_element_type=jnp.float32)

    n_tok = i1_ref.shape[0]
    lanes = lax.broadcasted_iota(jnp.int32, (n_tok, _E), 1)
    oh1 = (lanes == i1_ref[...]).astype(jnp.float32)
    oh2 = (lanes == i2_ref[...]).astype(jnp.float32)
    dn = (((1,), (0,)), ((), ()))
    base1 = lax.dot_general(oh1, off, dn, preferred_element_type=jnp.float32)
    base2 = lax.dot_general(oh2, off, dn, preferred_element_type=jnp.float32)
    s1_ref[...] = base1.astype(jnp.int32) + r1_ref[...]
    s2_ref[...] = base2.astype(jnp.int32) + r2_ref[...]

    ngid = gid_ref.shape[1]
    tst = (lax.broadcasted_iota(jnp.int32, (_E, ngid), 1) * _T)
    cmp = (off <= tst.astype(jnp.float32)).astype(jnp.float32)
    gid_ref[...] = jnp.sum(cmp, axis=0, keepdims=True).astype(jnp.int32) - 1


def _finalize(cnt_col, i1, i2, r1, r2, n_gid):
    n_tok = i1.shape[0]
    i32 = jnp.int32
    outs = [jax.ShapeDtypeStruct((n_tok, 1), i32),
            jax.ShapeDtypeStruct((n_tok, 1), i32),
            jax.ShapeDtypeStruct((1, n_gid), i32)]
    full = lambda s: pl.BlockSpec(s, lambda: (0,) * len(s))
    return pl.pallas_call(
        _finalize_body,
        in_specs=[full((_E, 1)), full((n_tok, 1)), full((n_tok, 1)),
                  full((n_tok, 1)), full((n_tok, 1))],
        out_specs=[full((n_tok, 1)), full((n_tok, 1)), full((1, n_gid))],
        out_shape=outs,
    )(cnt_col, i1, i2, r1, r2)


def _sc_scatter_x(xf, s1, s2, n_slots):
    n_tok, d = xf.shape
    per_w = n_tok // _NW
    nsub = per_w // _SUB
    mesh = plsc.VectorSubcoreMesh(core_axis_name="c", subcore_axis_name="s")

    @functools.partial(
        pl.kernel, mesh=mesh,
        out_type=jax.ShapeDtypeStruct((n_slots, d), jnp.float32),
        scratch_types=[pltpu.VMEM((per_w,), jnp.int32),
                       pltpu.VMEM((per_w,), jnp.int32),
                       pltpu.VMEM((_SUB, d), jnp.float32),
                       pltpu.SemaphoreType.DMA],
    )
    def k(x_hbm, s1_hbm, s2_hbm, xs_hbm, s1v, s2v, xv, sem):
        wid = lax.axis_index("s") * 2 + lax.axis_index("c")
        base = wid * per_w
        pltpu.sync_copy(s1_hbm.at[pl.ds(base, per_w)], s1v)
        pltpu.sync_copy(s2_hbm.at[pl.ds(base, per_w)], s2v)
        for j in range(nsub):
            pltpu.sync_copy(x_hbm.at[pl.ds(base + j * _SUB, _SUB)], xv)
            idx1 = s1v[pl.ds(j * _SUB, _SUB)]
            idx2 = s2v[pl.ds(j * _SUB, _SUB)]
            a = pltpu.async_copy(xv, xs_hbm.at[idx1], sem)
            b = pltpu.async_copy(xv, xs_hbm.at[idx2], sem)
            a.wait()
            b.wait()

    return k(xf, s1, s2)


def _sc_gather_buf(buf, s1, s2):
    n_slots, d = buf.shape
    n_tok = s1.shape[0]
    per_w = n_tok // _NW
    nsub = per_w // _SUB
    mesh = plsc.VectorSubcoreMesh(core_axis_name="c", subcore_axis_name="s")

    @functools.partial(
        pl.kernel, mesh=mesh,
        out_type=[jax.ShapeDtypeStruct((n_tok, d), jnp.float32),
                  jax.ShapeDtypeStruct((n_tok, d), jnp.float32)],
        scratch_types=[pltpu.VMEM((per_w,), jnp.int32),
                       pltpu.VMEM((per_w,), jnp.int32),
                       pltpu.VMEM((_SUB, d), jnp.float32),
                       pltpu.VMEM((_SUB, d), jnp.float32),
                       pltpu.SemaphoreType.DMA],
    )
    def k(buf_hbm, s1_hbm, s2_hbm, g1_hbm, g2_hbm, s1v, s2v, gv1, gv2, sem):
        wid = lax.axis_index("s") * 2 + lax.axis_index("c")
        base = wid * per_w
        pltpu.sync_copy(s1_hbm.at[pl.ds(base, per_w)], s1v)
        pltpu.sync_copy(s2_hbm.at[pl.ds(base, per_w)], s2v)
        for j in range(nsub):
            idx1 = s1v[pl.ds(j * _SUB, _SUB)]
            idx2 = s2v[pl.ds(j * _SUB, _SUB)]
            a = pltpu.async_copy(buf_hbm.at[idx1], gv1, sem)
            b = pltpu.async_copy(buf_hbm.at[idx2], gv2, sem)
            a.wait()
            b.wait()
            pltpu.sync_copy(gv1, g1_hbm.at[pl.ds(base + j * _SUB, _SUB)])
            pltpu.sync_copy(gv2, g2_hbm.at[pl.ds(base + j * _SUB, _SUB)])

    return k(buf, s1, s2)


def _gmm_body(gid_ref, xs_ref, we_ref, be_ref, out_ref):
    acc = lax.dot_general(xs_ref[...], we_ref[0], (((1,), (1,)), ((), ())),
                          preferred_element_type=jnp.float32)
    out_ref[...] = acc + be_ref[0]


def _grouped_matmul(gids, Xs, We, be3):
    n_slots, in_dim = Xs.shape
    out_dim = We.shape[1]
    n_tiles = n_slots // _T
    gspec = pltpu.PrefetchScalarGridSpec(
        num_scalar_prefetch=1,
        grid=(n_tiles,),
        in_specs=[
            pl.BlockSpec((_T, in_dim), lambda i, g: (i, 0)),
            pl.BlockSpec((1, out_dim, in_dim), lambda i, g: (g[i], 0, 0)),
            pl.BlockSpec((1, 1, out_dim), lambda i, g: (g[i], 0, 0)),
        ],
        out_specs=pl.BlockSpec((_T, out_dim), lambda i, g: (i, 0)),
    )
    return pl.pallas_call(
        _gmm_body,
        grid_spec=gspec,
        out_shape=jax.ShapeDtypeStruct((n_slots, out_dim), jnp.float32),
    )(gids, Xs, We, be3)


def _combine_body(g1_ref, g2_ref, w1_ref, w2_ref, out_ref):
    y = w1_ref[...] * g1_ref[...] + w2_ref[...] * g2_ref[...]
    out_ref[0] = y.T


_CTM = 512


def _combine(g1, g2, w1, w2, B, n_vars):
    n_tok, d = g1.shape
    tm = _CTM
    vpb = n_vars // tm
    tok_spec = pl.BlockSpec((tm, 1), lambda i: (i, 0))
    return pl.pallas_call(
        _combine_body,
        grid=(n_tok // tm,),
        in_specs=[pl.BlockSpec((tm, d), lambda i: (i, 0)),
                  pl.BlockSpec((tm, d), lambda i: (i, 0)),
                  tok_spec, tok_spec],
        out_specs=pl.BlockSpec((1, d, tm), lambda i: (i // vpb, 0, i % vpb)),
        out_shape=jax.ShapeDtypeStruct((B, d, n_vars), jnp.float32),
    )(g1, g2, w1, w2)


def kernel(x, Wg, We, be):
    B, in_len, n_vars = x.shape
    out_dim = We.shape[1]
    n_tok = B * n_vars
    n_slots = 2 * n_tok + _E * _T
    n_gid = max(64, n_slots // _T)

    xf = jnp.transpose(x, (0, 2, 1)).reshape(n_tok, in_len)
    w1, w2, i1, i2, r1, r2, cnt = _gating(xf, Wg)
    s1, s2, gids2 = _finalize(cnt.reshape(_E, 1), i1, i2, r1, r2, n_gid)
    s1f = s1.reshape(n_tok)
    s2f = s2.reshape(n_tok)
    Xs = _sc_scatter_x(xf, s1f, s2f, n_slots)
    be3 = be.reshape(_E, 1, out_dim)
    buf = _grouped_matmul(gids2.reshape(n_gid), Xs, We, be3)
    g1, g2 = _sc_gather_buf(buf, s1f, s2f)
    return _combine(g1, g2, w1, w2, B, n_vars)
